# baseline (speedup 1.0000x reference)
import functools

import jax
import jax.numpy as jnp
from jax import lax
from jax.experimental import pallas as pl
from jax.experimental.pallas import tpu as pltpu
from jax.experimental.pallas import tpu_sc as plsc

OFFSET = 2
HIDDEN = 2048
NC = 2
NS = 16
NW = NC * NS
CHUNK = 8


@functools.lru_cache(maxsize=None)
def _build(batch, seq):
    n_total = batch * seq
    b_per_w = n_total // NW
    n_chunk = b_per_w // CHUNK
    s_per_chunk = CHUNK // batch
    mesh = plsc.VectorSubcoreMesh(core_axis_name="c", subcore_axis_name="s")

    nbuf = 7

    @functools.partial(
        pl.kernel,
        out_type=jax.ShapeDtypeStruct((seq, batch, HIDDEN), jnp.float32),
        mesh=mesh,
        scratch_types=[
            pltpu.VMEM((b_per_w,), jnp.int32),
            pltpu.VMEM((nbuf, CHUNK, HIDDEN), jnp.float32),
            [pltpu.SemaphoreType.DMA] * nbuf,
            [pltpu.SemaphoreType.DMA] * nbuf,
        ],
    )
    def gather_kernel(idx_hbm, table_hbm, out_hbm, idx_v, rows_v, gsems, ssems):
        wid = lax.axis_index("s") * NC + lax.axis_index("c")
        s_base = wid * (b_per_w // batch)
        pltpu.sync_copy(idx_hbm.at[wid], idx_v)
        for k in range(b_per_w // 16):
            idx_v[pl.ds(k * 16, 16)] = idx_v[pl.ds(k * 16, 16)] + OFFSET

        ghandles = [None] * n_chunk
        shandles = [None] * n_chunk

        def start_gather(j):
            buf = j % nbuf
            ghandles[j] = pltpu.async_copy(
                table_hbm.at[idx_v.at[pl.ds(j * CHUNK, CHUNK)]],
                rows_v.at[buf],
                gsems[buf],
            )

        def start_store(j):
            buf = j % nbuf
            shandles[j] = [
                pltpu.async_copy(
                    rows_v.at[buf, pl.ds(i * batch, batch)],
                    out_hbm.at[s_base + j * s_per_chunk + i],
                    ssems[buf],
                )
                for i in range(s_per_chunk)
            ]

        for j in range(min(nbuf - 1, n_chunk)):
            start_gather(j)
        for j in range(n_chunk):
            ghandles[j].wait()
            start_store(j)
            nj = j + nbuf - 1
            if nj < n_chunk:
                if nj >= nbuf:
                    for h in shandles[nj - nbuf]:
                        h.wait()
                start_gather(nj)
        for j in range(n_chunk - nbuf, n_chunk):
            for h in shandles[j]:
                h.wait()

    return gather_kernel


def kernel(position_ids, weight):
    batch, seq = position_ids.shape
    n_total = batch * seq
    ids = jnp.transpose(position_ids, (1, 0)).astype(jnp.int32)
    idx = ids.reshape(NW, n_total // NW)
    return _build(batch, seq)(idx, weight)
